# baseline (speedup 1.0000x reference)
import jax
import jax.numpy as jnp
from jax.experimental import pallas as pl


def _subgraph_kernel(x_ref, adj_ref, w1_ref, b1_ref, w2_ref, b2_ref,
                     s_ref, id_ref, flag_ref):
    B, N, D = 8, 128, 128
    x = x_ref[:].reshape(B * 2 * N, D)
    h = jnp.maximum(
        jax.lax.dot_general(x, w1_ref[:], (((1,), (0,)), ((), ())),
                            preferred_element_type=jnp.float32) + b1_ref[:],
        0.0)
    s_all = jax.lax.dot_general(w2_ref[:], h, (((0,), (1,)), ((), ())),
                                preferred_element_type=jnp.float32) + b2_ref[:]

    s_rows = []
    a_rows = []
    for r in range(2):
        for b in range(B):
            seg = b * 2 + r
            s_rows.append(s_all[:, seg * N:(seg + 1) * N])
            a_rows.append(adj_ref[b, r:r + 1, :])
    ss = jnp.concatenate(s_rows, axis=0)
    aa = jnp.concatenate(a_rows, axis=0)

    rowid = jax.lax.broadcasted_iota(jnp.int32, (2 * B, 1), 0)
    col = jax.lax.broadcasted_iota(jnp.int32, (2 * B, N), 1)
    kill_col = jnp.where(rowid < B, 1, 0)
    aa = aa * (col != kill_col).astype(jnp.float32)
    sm = ss * aa

    for r in range(2):
        for b in range(B):
            g = r * B + b
            s_ref[b * 2 + r:b * 2 + r + 1, :] = sm[g:g + 1, :]

    mx = jnp.max(sm, axis=1, keepdims=True)
    cand = jnp.where(sm == mx, col, N)
    ids = jnp.min(cand, axis=1, keepdims=True)
    sub = ids[0:B]
    obj = ids[B:2 * B]
    id_ref[:, 0:1] = sub
    id_ref[:, 1:2] = obj
    a = sub > 0
    o = obj > 0
    flag_ref[:] = jnp.where(a & o, 3.0,
                            jnp.where(a, 1.0, jnp.where(o, 2.0, 0.0))
                            ).astype(jnp.float32)


def kernel(s_e, adjacency_matrix, W1, b1, W2, b2):
    B, N, _, D = s_e.shape
    out_shapes = (
        jax.ShapeDtypeStruct((2 * B, N), jnp.float32),
        jax.ShapeDtypeStruct((B, 2), jnp.int32),
        jax.ShapeDtypeStruct((B, 1), jnp.float32),
    )
    in_specs = [
        pl.BlockSpec((B, 2, N, D), lambda i: (0, 0, 0, 0)),
        pl.BlockSpec((B, 8, N), lambda i: (0, 0, 0)),
        pl.BlockSpec((D, D), lambda i: (0, 0)),
        pl.BlockSpec((1, D), lambda i: (0, 0)),
        pl.BlockSpec((D, 1), lambda i: (0, 0)),
        pl.BlockSpec((1, 1), lambda i: (0, 0)),
    ]
    out_specs = (
        pl.BlockSpec((2 * B, N), lambda i: (0, 0)),
        pl.BlockSpec((B, 2), lambda i: (0, 0)),
        pl.BlockSpec((B, 1), lambda i: (0, 0)),
    )
    scores, ids, flag = pl.pallas_call(
        _subgraph_kernel,
        grid=(1,),
        in_specs=in_specs,
        out_specs=out_specs,
        out_shape=out_shapes,
    )(s_e, adjacency_matrix, W1, b1.reshape(1, D), W2, b2.reshape(1, 1))

    return ids, scores.reshape(B, 2, N), flag.reshape(B)
